# Initial kernel scaffold; baseline (speedup 1.0000x reference)
import jax
import jax.numpy as jnp
from jax.experimental import pallas as pl


def kernel(x, edge_index_connections, edge_index_destinations, W_l1, W_r1, b1, W_l2, W_r2, b2, W_l3, W_r3, b3):
    raise NotImplementedError("write your pallas kernel here")


# speedup vs baseline: 8.9508x; 8.9508x over previous
import functools

import jax
import jax.numpy as jnp
from jax import lax
from jax.experimental import pallas as pl
from jax.experimental.pallas import tpu as pltpu
from jax.experimental.pallas import tpu_sc as plsc

N = 100000
HIDDEN = 32
IN_DIM = 8

NC = 2
NS = 16
NW = NC * NS

PAD_ROWS = 1120
N2 = N + PAD_ROWS
RPT = N2 // NS

E1 = 1600000
E1P = 1638400
E2 = 100000
E2P = 102400


def _agg_edge_split(table, src2, dst2, zeros2):
    WR = 8
    CHR = E1P // 128 // NW
    NWIN = CHR // WR

    def body(table_h, src_h, dst_h, zeros_h, out_h, sidx, didx, rows, acc, sem):
        c = lax.axis_index("c")
        s = lax.axis_index("s")
        wid = s * NC + c
        r0 = s * RPT
        pltpu.sync_copy(zeros_h.at[pl.ds(r0, RPT)], acc.at[pl.ds(r0, RPT)])
        plsc.subcore_barrier()
        row0 = wid * CHR

        def w_body(w, carry):
            wr0 = row0 + w * WR
            pltpu.sync_copy(src_h.at[pl.ds(wr0, WR)], sidx)
            pltpu.sync_copy(dst_h.at[pl.ds(wr0, WR)], didx)
            descs = [pltpu.async_copy(table_h.at[sidx.at[j]], rows.at[j], sem)
                     for j in range(WR)]
            for d in descs:
                d.wait()
            for j in range(WR):
                pltpu.sync_copy(rows.at[j], acc.at[didx.at[j]], add=True)
            return carry

        lax.fori_loop(0, NWIN, w_body, 0)
        plsc.subcore_barrier()
        pltpu.sync_copy(acc.at[pl.ds(r0, RPT)], out_h.at[c, pl.ds(r0, RPT)])

    fn = pl.kernel(
        body,
        out_type=jax.ShapeDtypeStruct((NC, N2, 16), jnp.float32),
        mesh=plsc.VectorSubcoreMesh(core_axis_name="c", subcore_axis_name="s"),
        compiler_params=pltpu.CompilerParams(use_tc_tiling_on_sc=False),
        scratch_types=[
            pltpu.VMEM((WR, 128), jnp.int32),
            pltpu.VMEM((WR, 128), jnp.int32),
            pltpu.VMEM((WR, 128, 16), jnp.float32),
            pltpu.VMEM_SHARED((N2, 16), jnp.float32),
            pltpu.SemaphoreType.DMA,
        ],
    )
    return fn(table, src2, dst2, zeros2)


def _agg_col_split(table, src2, dst2, zeros2, zeros1, ep, with_count):
    CHR = ep // 128 // NS
    WR = 8 if CHR % 8 == 0 else 5
    NWIN = CHR // WR
    assert WR * NWIN == CHR
    CHRC = ep // 128 // NW
    WRC = 5 if with_count else 1
    NWINC = CHRC // WRC if with_count else 0
    assert (not with_count) or WRC * NWINC == CHRC

    def body(table_h, src_h, dst_h, zeros_h, zeros1_h, agg_h, cnt_h,
             sidx, didx, rows, ones, acc, cnt, sem):
        c = lax.axis_index("c")
        s = lax.axis_index("s")
        r0 = s * RPT
        pltpu.sync_copy(zeros_h.at[pl.ds(r0, RPT)], acc.at[pl.ds(r0, RPT)])
        if with_count:
            pltpu.sync_copy(zeros1_h.at[pl.ds(r0, RPT)], cnt.at[pl.ds(r0, RPT)])
            for j in range(WRC):
                for k in range(8):
                    ones[j, pl.ds(k * 16, 16)] = jnp.ones((16,), jnp.float32)
        plsc.subcore_barrier()

        tbl = table_h.at[c]
        row0 = s * CHR

        def w_body(w, carry):
            wr0 = row0 + w * WR
            pltpu.sync_copy(src_h.at[pl.ds(wr0, WR)], sidx.at[pl.ds(0, WR)])
            pltpu.sync_copy(dst_h.at[pl.ds(wr0, WR)], didx.at[pl.ds(0, WR)])
            descs = [pltpu.async_copy(tbl.at[sidx.at[j]], rows.at[j], sem)
                     for j in range(WR)]
            for d in descs:
                d.wait()
            for j in range(WR):
                pltpu.sync_copy(rows.at[j], acc.at[didx.at[j]], add=True)
            return carry

        lax.fori_loop(0, NWIN, w_body, 0)

        if with_count:
            wid = s * NC + c
            crow0 = wid * CHRC

            def c_body(w, carry):
                wr0 = crow0 + w * WRC
                pltpu.sync_copy(dst_h.at[pl.ds(wr0, WRC)], didx.at[pl.ds(0, WRC)])
                for j in range(WRC):
                    pltpu.sync_copy(ones.at[j], cnt.at[didx.at[j]], add=True)
                return carry

            lax.fori_loop(0, NWINC, c_body, 0)

        plsc.subcore_barrier()
        pltpu.sync_copy(acc.at[pl.ds(r0, RPT)], agg_h.at[c, pl.ds(r0, RPT)])
        if with_count:
            pltpu.sync_copy(cnt.at[pl.ds(r0, RPT)], cnt_h.at[c, pl.ds(r0, RPT)])

    fn = pl.kernel(
        body,
        out_type=(
            jax.ShapeDtypeStruct((NC, N2, 16), jnp.float32),
            jax.ShapeDtypeStruct((NC, N2), jnp.float32),
        ),
        mesh=plsc.VectorSubcoreMesh(core_axis_name="c", subcore_axis_name="s"),
        compiler_params=pltpu.CompilerParams(use_tc_tiling_on_sc=False),
        scratch_types=[
            pltpu.VMEM((max(WR, WRC), 128), jnp.int32),
            pltpu.VMEM((max(WR, WRC), 128), jnp.int32),
            pltpu.VMEM((WR, 128, 16), jnp.float32),
            pltpu.VMEM((WRC, 128), jnp.float32),
            pltpu.VMEM_SHARED((N2, 16), jnp.float32),
            pltpu.VMEM_SHARED((N2,) if with_count else (128,), jnp.float32),
            pltpu.SemaphoreType.DMA,
        ],
    )
    return fn(table, src2, dst2, zeros2, zeros1)


BLK = 4000
GRID = N // BLK


def _dense1(parts, x_pad, wl, wr, b):
    def body(p_ref, x_ref, wl_ref, wr_ref, b_ref, h_ref, cnt_ref):
        agg = p_ref[0] + p_ref[1]
        cnt = agg[:, 8:9]
        mean = agg / jnp.maximum(cnt, 1.0)
        out = (jnp.dot(mean, wl_ref[...], preferred_element_type=jnp.float32)
               + jnp.dot(x_ref[...], wr_ref[...], preferred_element_type=jnp.float32)
               + b_ref[...])
        nrm = jnp.sqrt(jnp.sum(out * out, axis=1, keepdims=True))
        out = out / jnp.maximum(nrm, 1e-12)
        out = jnp.tanh(out)
        h_ref[0] = out[:, :16]
        h_ref[1] = out[:, 16:]
        cnt_ref[...] = cnt

    return pl.pallas_call(
        body,
        grid=(GRID,),
        in_specs=[
            pl.BlockSpec((2, BLK, 16), lambda i: (0, i, 0)),
            pl.BlockSpec((BLK, 16), lambda i: (i, 0)),
            pl.BlockSpec((16, HIDDEN), lambda i: (0, 0)),
            pl.BlockSpec((16, HIDDEN), lambda i: (0, 0)),
            pl.BlockSpec((1, HIDDEN), lambda i: (0, 0)),
        ],
        out_specs=[
            pl.BlockSpec((2, BLK, 16), lambda i: (0, i, 0)),
            pl.BlockSpec((BLK, 1), lambda i: (i, 0)),
        ],
        out_shape=[
            jax.ShapeDtypeStruct((2, N, 16), jnp.float32),
            jax.ShapeDtypeStruct((N, 1), jnp.float32),
        ],
    )(parts, x_pad, wl, wr, b)


def _dense23(agg, cnt, h, wl_a, wl_b, wr_a, wr_b, b, final):
    cnt_is_split = cnt.ndim == 3

    def body(agg_ref, cnt_ref, h_ref, wla_ref, wlb_ref, wra_ref, wrb_ref,
             b_ref, o_ref):
        if cnt_is_split:
            cnt = cnt_ref[0] + cnt_ref[1]
        else:
            cnt = cnt_ref[...]
        cnt = jnp.maximum(cnt, 1.0)
        mean0 = agg_ref[0] / cnt
        mean1 = agg_ref[1] / cnt
        out = (jnp.dot(mean0, wla_ref[...], preferred_element_type=jnp.float32)
               + jnp.dot(mean1, wlb_ref[...], preferred_element_type=jnp.float32)
               + jnp.dot(h_ref[0], wra_ref[...], preferred_element_type=jnp.float32)
               + jnp.dot(h_ref[1], wrb_ref[...], preferred_element_type=jnp.float32)
               + b_ref[...])
        nrm = jnp.sqrt(jnp.sum(out * out, axis=1, keepdims=True))
        out = out / jnp.maximum(nrm, 1e-12)
        out = jnp.tanh(out)
        if final:
            o_ref[...] = out
        else:
            o_ref[0] = out[:, :16]
            o_ref[1] = out[:, 16:]

    cnt_spec = (pl.BlockSpec((2, BLK, 1), lambda i: (0, i, 0)) if cnt_is_split
                else pl.BlockSpec((BLK, 1), lambda i: (i, 0)))
    out_spec = (pl.BlockSpec((BLK, HIDDEN), lambda i: (i, 0)) if final
                else pl.BlockSpec((2, BLK, 16), lambda i: (0, i, 0)))
    out_shape = (jax.ShapeDtypeStruct((N, HIDDEN), jnp.float32) if final
                 else jax.ShapeDtypeStruct((2, N, 16), jnp.float32))
    return pl.pallas_call(
        body,
        grid=(GRID,),
        in_specs=[
            pl.BlockSpec((2, BLK, 16), lambda i: (0, i, 0)),
            cnt_spec,
            pl.BlockSpec((2, BLK, 16), lambda i: (0, i, 0)),
            pl.BlockSpec((16, HIDDEN), lambda i: (0, 0)),
            pl.BlockSpec((16, HIDDEN), lambda i: (0, 0)),
            pl.BlockSpec((16, HIDDEN), lambda i: (0, 0)),
            pl.BlockSpec((16, HIDDEN), lambda i: (0, 0)),
            pl.BlockSpec((1, HIDDEN), lambda i: (0, 0)),
        ],
        out_specs=out_spec,
        out_shape=out_shape,
    )(agg, cnt, h, wl_a, wl_b, wr_a, wr_b, b)


def _pad_edges(edge_index, ep):
    src = edge_index[0]
    dst = edge_index[1]
    npad = ep - src.shape[0]
    pad_src = (jnp.arange(npad, dtype=jnp.int32) * 97) % N
    pad_dst = N + (jnp.arange(npad, dtype=jnp.int32) % PAD_ROWS)
    src2 = jnp.concatenate([src, pad_src]).reshape(ep // 128, 128)
    dst2 = jnp.concatenate([dst, pad_dst]).reshape(ep // 128, 128)
    return src2, dst2


def kernel(x, edge_index_connections, edge_index_destinations,
           W_l1, W_r1, b1, W_l2, W_r2, b2, W_l3, W_r3, b3):
    f32 = jnp.float32
    src1, dst1 = _pad_edges(edge_index_connections, E1P)
    src2, dst2 = _pad_edges(edge_index_destinations, E2P)

    x_pad = jnp.zeros((N, 16), f32).at[:, :IN_DIM].set(x).at[:, IN_DIM].set(1.0)
    zeros2 = jnp.zeros((N2, 16), f32)
    zeros1 = jnp.zeros((N2,), f32)

    wl1 = jnp.zeros((16, HIDDEN), f32).at[:IN_DIM].set(W_l1)
    wr1 = jnp.zeros((16, HIDDEN), f32).at[:IN_DIM].set(W_r1)
    wl2_a, wl2_b = W_l2[:16], W_l2[16:]
    wr2_a, wr2_b = W_r2[:16], W_r2[16:]
    wl3_a, wl3_b = W_l3[:16], W_l3[16:]
    wr3_a, wr3_b = W_r3[:16], W_r3[16:]
    b1r = b1.reshape(1, HIDDEN)
    b2r = b2.reshape(1, HIDDEN)
    b3r = b3.reshape(1, HIDDEN)

    parts = _agg_edge_split(x_pad, src1, dst1, zeros2)
    h1, cnt1 = _dense1(parts, x_pad, wl1, wr1, b1r)

    agg2, cnt2p = _agg_col_split(h1, src2, dst2, zeros2, zeros1, E2P, True)
    h2 = _dense23(agg2, cnt2p.reshape(NC, N2, 1), h1,
                  wl2_a, wl2_b, wr2_a, wr2_b, b2r, False)

    agg3, _ = _agg_col_split(h2, src1, dst1, zeros2, zeros1, E1P, False)
    out = _dense23(agg3, cnt1, h2, wl3_a, wl3_b, wr3_a, wr3_b, b3r, True)
    return out


# speedup vs baseline: 16.8405x; 1.8814x over previous
import numpy as np
import jax
import jax.numpy as jnp
from jax import lax
from jax.experimental import pallas as pl
from jax.experimental.pallas import tpu as pltpu
from jax.experimental.pallas import tpu_sc as plsc

N = 100000
HIDDEN = 32
IN_DIM = 8

NC = 2
NS = 16
NW = NC * NS

PAD_ROWS = 1120
N2 = N + PAD_ROWS
RPT = N2 // NS
NP8 = N // 8
N2P8 = N2 // 8

E1 = 1600000
E1P = 1638400
E2 = 100000
E2P = 102400


def _agg_edge_split(table, src2, dst2, zeros2):
    WR = 8
    CHR = E1P // 128 // NW
    NWIN = CHR // WR

    def body(table_h, src_h, dst_h, zeros_h, out_h, sidx, didx, rows, acc, sem):
        c = lax.axis_index("c")
        s = lax.axis_index("s")
        wid = s * NC + c
        r0 = s * RPT
        pltpu.sync_copy(zeros_h.at[pl.ds(r0, RPT)], acc.at[pl.ds(r0, RPT)])
        plsc.subcore_barrier()
        row0 = wid * CHR

        def w_body(w, carry):
            wr0 = row0 + w * WR
            pltpu.sync_copy(src_h.at[pl.ds(wr0, WR)], sidx)
            pltpu.sync_copy(dst_h.at[pl.ds(wr0, WR)], didx)
            descs = [pltpu.async_copy(table_h.at[sidx.at[j]], rows.at[j], sem)
                     for j in range(WR)]
            for d in descs:
                d.wait()
            for j in range(WR):
                pltpu.sync_copy(rows.at[j], acc.at[didx.at[j]], add=True)
            return carry

        lax.fori_loop(0, NWIN, w_body, 0)
        plsc.subcore_barrier()
        pltpu.sync_copy(acc.at[pl.ds(r0, RPT)], out_h.at[c, pl.ds(r0, RPT)])

    fn = pl.kernel(
        body,
        out_type=jax.ShapeDtypeStruct((NC, N2, 16), jnp.float32),
        mesh=plsc.VectorSubcoreMesh(core_axis_name="c", subcore_axis_name="s"),
        compiler_params=pltpu.CompilerParams(use_tc_tiling_on_sc=False),
        scratch_types=[
            pltpu.VMEM((WR, 128), jnp.int32),
            pltpu.VMEM((WR, 128), jnp.int32),
            pltpu.VMEM((WR, 128, 16), jnp.float32),
            pltpu.VMEM_SHARED((N2, 16), jnp.float32),
            pltpu.SemaphoreType.DMA,
        ],
    )
    return fn(table, src2, dst2, zeros2)


def _agg_col_split(table, src2, dst2, zeros2, ep, with_count):
    CHR = ep // 128 // NS
    WR = 8 if CHR % 8 == 0 else 5
    NWIN = CHR // WR
    assert WR * NWIN == CHR
    CHRC = ep // 128 // NW
    WRC = 5 if with_count else 1
    NWINC = CHRC // WRC if with_count else 0
    assert (not with_count) or WRC * NWINC == CHRC

    def body(table_h, src_h, dst_h, zeros_h, agg_h, cnt_h,
             sidx, didx, rows, ones, acc, sem):
        c = lax.axis_index("c")
        s = lax.axis_index("s")
        r0 = s * RPT
        pltpu.sync_copy(zeros_h.at[pl.ds(r0, RPT)], acc.at[pl.ds(r0, RPT)])
        if with_count:
            for k in range(128):
                ones[k, pl.ds(0, 16)] = jnp.ones((16,), jnp.float32)
        plsc.subcore_barrier()

        tbl = table_h.at[c]
        row0 = s * CHR

        def w_body(w, carry):
            wr0 = row0 + w * WR
            pltpu.sync_copy(src_h.at[pl.ds(wr0, WR)], sidx.at[pl.ds(0, WR)])
            pltpu.sync_copy(dst_h.at[pl.ds(wr0, WR)], didx.at[pl.ds(0, WR)])
            descs = [pltpu.async_copy(tbl.at[sidx.at[j]], rows.at[j], sem)
                     for j in range(WR)]
            for d in descs:
                d.wait()
            for j in range(WR):
                pltpu.sync_copy(rows.at[j], acc.at[didx.at[j]], add=True)
            return carry

        lax.fori_loop(0, NWIN, w_body, 0)
        plsc.subcore_barrier()
        pltpu.sync_copy(acc.at[pl.ds(r0, RPT)], agg_h.at[c, pl.ds(r0, RPT)])

        if with_count:
            plsc.subcore_barrier()
            pltpu.sync_copy(zeros_h.at[pl.ds(r0, RPT)], acc.at[pl.ds(r0, RPT)])
            plsc.subcore_barrier()
            wid = s * NC + c
            crow0 = wid * CHRC

            def c_body(w, carry):
                wr0 = crow0 + w * WRC
                pltpu.sync_copy(dst_h.at[pl.ds(wr0, WRC)],
                                didx.at[pl.ds(0, WRC)])
                for j in range(WRC):
                    pltpu.sync_copy(ones, acc.at[didx.at[j]], add=True)
                return carry

            lax.fori_loop(0, NWINC, c_body, 0)
            plsc.subcore_barrier()
            pltpu.sync_copy(acc.at[pl.ds(r0, RPT)], cnt_h.at[c, pl.ds(r0, RPT)])

    fn = pl.kernel(
        body,
        out_type=(
            jax.ShapeDtypeStruct((NC, N2, 16), jnp.float32),
            jax.ShapeDtypeStruct((NC, N2, 16) if with_count else (NC, 8, 16),
                                 jnp.float32),
        ),
        mesh=plsc.VectorSubcoreMesh(core_axis_name="c", subcore_axis_name="s"),
        compiler_params=pltpu.CompilerParams(use_tc_tiling_on_sc=False),
        scratch_types=[
            pltpu.VMEM((WR, 128), jnp.int32),
            pltpu.VMEM((WR, 128), jnp.int32),
            pltpu.VMEM((WR, 128, 16), jnp.float32),
            pltpu.VMEM((128, 16), jnp.float32),
            pltpu.VMEM_SHARED((N2, 16), jnp.float32),
            pltpu.SemaphoreType.DMA,
        ],
    )
    return fn(table, src2, dst2, zeros2)


GRID = 4
BR = N2P8 // GRID
BLKN = BR * 8


def _np_kron_eye(w, k):
    r, c = w.shape
    out = jnp.zeros((k * r, k * c), jnp.float32)
    for a in range(k):
        out = lax.dynamic_update_slice(out, w, (a * r, a * c))
    return out


def _sel_count():
    s = np.zeros((128, 128), np.float32)
    for a in range(8):
        s[16 * a + 8, 16 * a:16 * a + 16] = 1.0
    return s


def _sel_gsum():
    g = np.zeros((256, 256), np.float32)
    for a in range(8):
        g[32 * a:32 * a + 32, 32 * a:32 * a + 32] = 1.0
    return g


def _sel_half(which):
    p = np.zeros((256, 128), np.float32)
    for a in range(8):
        for k in range(16):
            p[32 * a + 16 * which + k, 16 * a + k] = 1.0
    return p


_S_CNT = _sel_count()
_G_SUM = _sel_gsum()
_P_HALF0 = _sel_half(0)
_P_HALF1 = _sel_half(1)


def _sel_pack(which):
    t = np.zeros((128, 128), np.float32)
    for m in range(8):
        for k in range(IN_DIM):
            t[64 * which + 8 * m + k, 16 * m + k] = 1.0
    return t


_T_PACK0 = _sel_pack(0)
_T_PACK1 = _sel_pack(1)
_ONES_LANE = np.zeros((1, 128), np.float32)
for _m in range(8):
    _ONES_LANE[0, 16 * _m + IN_DIM] = 1.0


def _prep_x(x):
    x_lin = x.reshape(N // 16, 128)
    a = x_lin @ jnp.asarray(_T_PACK0)
    b = x_lin @ jnp.asarray(_T_PACK1)
    pk = jnp.stack([a, b], axis=1).reshape(NP8, 128) + jnp.asarray(_ONES_LANE)
    return jnp.concatenate(
        [pk, jnp.zeros((N2P8 - NP8, 128), jnp.float32)], axis=0)


def _dense1(parts_pk, x_pk, kwl, kwr, kb, gsum, scnt, ph0, ph1):
    def body(p_ref, x_ref, kwl_ref, kwr_ref, kb_ref, g_ref, s_ref,
             ph0_ref, ph1_ref, h_ref, cnt_ref):
        agg = p_ref[0] + p_ref[1]
        cnt = jnp.dot(agg, s_ref[...], preferred_element_type=jnp.float32)
        mean = agg / jnp.maximum(cnt, 1.0)
        out = (jnp.dot(mean, kwl_ref[...], preferred_element_type=jnp.float32)
               + jnp.dot(x_ref[...], kwr_ref[...],
                         preferred_element_type=jnp.float32)
               + kb_ref[...])
        ss = jnp.dot(out * out, g_ref[...], preferred_element_type=jnp.float32)
        out = out / jnp.maximum(jnp.sqrt(ss), 1e-12)
        out = jnp.tanh(out)
        h_ref[0] = jnp.dot(out, ph0_ref[...],
                           preferred_element_type=jnp.float32)
        h_ref[1] = jnp.dot(out, ph1_ref[...],
                           preferred_element_type=jnp.float32)
        cnt_ref[...] = cnt

    return pl.pallas_call(
        body,
        grid=(GRID,),
        in_specs=[
            pl.BlockSpec((2, BR, 128), lambda i: (0, i, 0)),
            pl.BlockSpec((BR, 128), lambda i: (i, 0)),
            pl.BlockSpec((128, 256), lambda i: (0, 0)),
            pl.BlockSpec((128, 256), lambda i: (0, 0)),
            pl.BlockSpec((1, 256), lambda i: (0, 0)),
            pl.BlockSpec((256, 256), lambda i: (0, 0)),
            pl.BlockSpec((128, 128), lambda i: (0, 0)),
            pl.BlockSpec((256, 128), lambda i: (0, 0)),
            pl.BlockSpec((256, 128), lambda i: (0, 0)),
        ],
        out_specs=[
            pl.BlockSpec((2, BR, 128), lambda i: (0, i, 0)),
            pl.BlockSpec((BR, 128), lambda i: (i, 0)),
        ],
        out_shape=[
            jax.ShapeDtypeStruct((2, N2P8, 128), jnp.float32),
            jax.ShapeDtypeStruct((N2P8, 128), jnp.float32),
        ],
    )(parts_pk, x_pk, kwl, kwr, kb, gsum, scnt, ph0, ph1)


def _dense23(agg_pk, cnt_pk, h_pk, kwl_a, kwl_b, kwr_a, kwr_b, kb, gsum,
             ph0, ph1, final):
    cnt_is_split = cnt_pk.ndim == 3

    def body(agg_ref, cnt_ref, h_ref, kwla_ref, kwlb_ref, kwra_ref, kwrb_ref,
             kb_ref, g_ref, ph0_ref, ph1_ref, o_ref):
        if cnt_is_split:
            cnt = cnt_ref[0] + cnt_ref[1]
        else:
            cnt = cnt_ref[...]
        cnt = jnp.maximum(cnt, 1.0)
        mean0 = agg_ref[0] / cnt
        mean1 = agg_ref[1] / cnt
        out = (jnp.dot(mean0, kwla_ref[...], preferred_element_type=jnp.float32)
               + jnp.dot(mean1, kwlb_ref[...],
                         preferred_element_type=jnp.float32)
               + jnp.dot(h_ref[0], kwra_ref[...],
                         preferred_element_type=jnp.float32)
               + jnp.dot(h_ref[1], kwrb_ref[...],
                         preferred_element_type=jnp.float32)
               + kb_ref[...])
        ss = jnp.dot(out * out, g_ref[...], preferred_element_type=jnp.float32)
        out = out / jnp.maximum(jnp.sqrt(ss), 1e-12)
        out = jnp.tanh(out)
        if final:
            o_ref[...] = out
        else:
            o_ref[0] = jnp.dot(out, ph0_ref[...],
                               preferred_element_type=jnp.float32)
            o_ref[1] = jnp.dot(out, ph1_ref[...],
                               preferred_element_type=jnp.float32)

    cnt_spec = (pl.BlockSpec((2, BR, 128), lambda i: (0, i, 0)) if cnt_is_split
                else pl.BlockSpec((BR, 128), lambda i: (i, 0)))
    out_spec = (pl.BlockSpec((BR, 256), lambda i: (i, 0)) if final
                else pl.BlockSpec((2, BR, 128), lambda i: (0, i, 0)))
    out_shape = (jax.ShapeDtypeStruct((N2P8, 256), jnp.float32) if final
                 else jax.ShapeDtypeStruct((2, N2P8, 128), jnp.float32))
    return pl.pallas_call(
        body,
        grid=(GRID,),
        in_specs=[
            pl.BlockSpec((2, BR, 128), lambda i: (0, i, 0)),
            cnt_spec,
            pl.BlockSpec((2, BR, 128), lambda i: (0, i, 0)),
            pl.BlockSpec((128, 256), lambda i: (0, 0)),
            pl.BlockSpec((128, 256), lambda i: (0, 0)),
            pl.BlockSpec((128, 256), lambda i: (0, 0)),
            pl.BlockSpec((128, 256), lambda i: (0, 0)),
            pl.BlockSpec((1, 256), lambda i: (0, 0)),
            pl.BlockSpec((256, 256), lambda i: (0, 0)),
            pl.BlockSpec((256, 128), lambda i: (0, 0)),
            pl.BlockSpec((256, 128), lambda i: (0, 0)),
        ],
        out_specs=out_spec,
        out_shape=out_shape,
    )(agg_pk, cnt_pk, h_pk, kwl_a, kwl_b, kwr_a, kwr_b, kb, gsum, ph0, ph1)


def _pad_consts(e, ep):
    npad = ep - e
    pad_src = ((np.arange(npad, dtype=np.int64) * 97) % N).astype(np.int32)
    pad_dst = (N + (np.arange(npad, dtype=np.int64) % PAD_ROWS)).astype(
        np.int32)
    return pad_src, pad_dst


_PAD1 = _pad_consts(E1, E1P)
_PAD2 = _pad_consts(E2, E2P)


def _pad_edges(edge_index, ep, pads):
    pad_src, pad_dst = pads
    src2 = jnp.concatenate([edge_index[0], pad_src]).reshape(ep // 128, 128)
    dst2 = jnp.concatenate([edge_index[1], pad_dst]).reshape(ep // 128, 128)
    return src2, dst2


def kernel(x, edge_index_connections, edge_index_destinations,
           W_l1, W_r1, b1, W_l2, W_r2, b2, W_l3, W_r3, b3):
    f32 = jnp.float32
    src1, dst1 = _pad_edges(edge_index_connections, E1P, _PAD1)
    src2, dst2 = _pad_edges(edge_index_destinations, E2P, _PAD2)

    zeros2 = jnp.zeros((N2, 16), f32)
    gsum = jnp.asarray(_G_SUM)
    scnt = jnp.asarray(_S_CNT)
    ph0 = jnp.asarray(_P_HALF0)
    ph1 = jnp.asarray(_P_HALF1)

    wl1 = jnp.zeros((16, HIDDEN), f32).at[:IN_DIM].set(W_l1)
    wr1 = jnp.zeros((16, HIDDEN), f32).at[:IN_DIM].set(W_r1)
    kwl1 = _np_kron_eye(wl1, 8)
    kwr1 = _np_kron_eye(wr1, 8)
    kwl2_a = _np_kron_eye(W_l2[:16], 8)
    kwl2_b = _np_kron_eye(W_l2[16:], 8)
    kwr2_a = _np_kron_eye(W_r2[:16], 8)
    kwr2_b = _np_kron_eye(W_r2[16:], 8)
    kwl3_a = _np_kron_eye(W_l3[:16], 8)
    kwl3_b = _np_kron_eye(W_l3[16:], 8)
    kwr3_a = _np_kron_eye(W_r3[:16], 8)
    kwr3_b = _np_kron_eye(W_r3[16:], 8)
    kb1 = jnp.tile(b1, 8).reshape(1, 256)
    kb2 = jnp.tile(b2, 8).reshape(1, 256)
    kb3 = jnp.tile(b3, 8).reshape(1, 256)

    x_pk = _prep_x(x)
    x_tbl = x_pk.reshape(N2, 16)

    parts = _agg_edge_split(x_tbl, src1, dst1, zeros2)
    h1, cnt1 = _dense1(parts.reshape(NC, N2P8, 128), x_pk,
                       kwl1, kwr1, kb1, gsum, scnt, ph0, ph1)

    agg2, cnt2p = _agg_col_split(h1.reshape(NC, N2, 16), src2, dst2, zeros2,
                                 E2P, True)
    h2 = _dense23(agg2.reshape(NC, N2P8, 128), cnt2p.reshape(NC, N2P8, 128),
                  h1, kwl2_a, kwl2_b, kwr2_a, kwr2_b, kb2, gsum, ph0, ph1,
                  False)

    agg3, _ = _agg_col_split(h2.reshape(NC, N2, 16), src1, dst1, zeros2,
                             E1P, False)
    out = _dense23(agg3.reshape(NC, N2P8, 128), cnt1, h2,
                   kwl3_a, kwl3_b, kwr3_a, kwr3_b, kb3, gsum, ph0, ph1,
                   True)
    return out[:NP8].reshape(N, HIDDEN)


# speedup vs baseline: 20.4581x; 1.2148x over previous
import numpy as np
import jax
import jax.numpy as jnp
from jax import lax
from jax.experimental import pallas as pl
from jax.experimental.pallas import tpu as pltpu
from jax.experimental.pallas import tpu_sc as plsc

N = 100000
HIDDEN = 32
IN_DIM = 8

NC = 2
NS = 16
NW = NC * NS

PAD_ROWS = 1120
N2 = N + PAD_ROWS
RPT = N2 // NS
NP8 = N // 8
N2P8 = N2 // 8

E1 = 1600000
E1P = 1638400
E2 = 100000
E2P = 102400


def _agg_edge_split(table, src2, dst2, zeros2):
    WR = 5
    CHR = E1P // 128 // NW
    NWIN = CHR // WR
    NG = NWIN // 2

    def body(table_h, src_h, dst_h, zeros_h, out_h, sidx, didx, rows, acc,
             sem0, sem1):
        c = lax.axis_index("c")
        s = lax.axis_index("s")
        wid = s * NC + c
        r0 = s * RPT
        pltpu.sync_copy(zeros_h.at[pl.ds(r0, RPT)], acc.at[pl.ds(r0, RPT)])
        plsc.subcore_barrier()
        row0 = wid * CHR
        sems = (sem0, sem1)
        dummy = table_h.at[pl.ds(0, 128)]

        def load_issue(w, b):
            wr0 = row0 + w * WR
            pltpu.sync_copy(src_h.at[pl.ds(wr0, WR)], sidx.at[b])
            pltpu.sync_copy(dst_h.at[pl.ds(wr0, WR)], didx.at[b])
            for j in range(WR):
                pltpu.async_copy(table_h.at[sidx.at[b, j]], rows.at[b, j],
                                 sems[b])

        load_issue(0, 0)
        load_issue(1, 1)

        def g_body(g, carry):
            for b in range(2):
                for j in range(WR):
                    pltpu.make_async_copy(dummy, rows.at[b, j], sems[b]).wait()
                for j in range(WR):
                    pltpu.sync_copy(rows.at[b, j], acc.at[didx.at[b, j]],
                                    add=True)

                @pl.when(g < NG - 1)
                def _():
                    load_issue(2 * g + b + 2, b)
            return carry

        lax.fori_loop(0, NG, g_body, 0)
        plsc.subcore_barrier()
        pltpu.sync_copy(acc.at[pl.ds(r0, RPT)], out_h.at[c, pl.ds(r0, RPT)])

    fn = pl.kernel(
        body,
        out_type=jax.ShapeDtypeStruct((NC, N2, 16), jnp.float32),
        mesh=plsc.VectorSubcoreMesh(core_axis_name="c", subcore_axis_name="s"),
        compiler_params=pltpu.CompilerParams(use_tc_tiling_on_sc=False),
        scratch_types=[
            pltpu.VMEM((2, WR, 128), jnp.int32),
            pltpu.VMEM((2, WR, 128), jnp.int32),
            pltpu.VMEM((2, WR, 128, 16), jnp.float32),
            pltpu.VMEM_SHARED((N2, 16), jnp.float32),
            pltpu.SemaphoreType.DMA,
            pltpu.SemaphoreType.DMA,
        ],
    )
    return fn(table, src2, dst2, zeros2)


def _agg_col_split(table, src2, dst2, zeros2, ep, with_count):
    CHR = ep // 128 // NS
    WR = 5
    NWIN = CHR // WR
    assert WR * NWIN == CHR and NWIN % 2 == 0
    NG = NWIN // 2
    CHRC = ep // 128 // NW
    WRC = 5 if with_count else 1
    NWINC = CHRC // WRC if with_count else 0
    assert (not with_count) or WRC * NWINC == CHRC

    def body(table_h, src_h, dst_h, zeros_h, agg_h, cnt_h,
             sidx, didx, rows, ones, acc, sem0, sem1):
        c = lax.axis_index("c")
        s = lax.axis_index("s")
        r0 = s * RPT
        pltpu.sync_copy(zeros_h.at[pl.ds(r0, RPT)], acc.at[pl.ds(r0, RPT)])
        if with_count:
            for k in range(128):
                ones[k, pl.ds(0, 16)] = jnp.ones((16,), jnp.float32)
        plsc.subcore_barrier()

        tbl = table_h.at[c]
        row0 = s * CHR
        sems = (sem0, sem1)
        dummy = table_h.at[0, pl.ds(0, 128)]

        def load_issue(w, b):
            wr0 = row0 + w * WR
            pltpu.sync_copy(src_h.at[pl.ds(wr0, WR)], sidx.at[b])
            pltpu.sync_copy(dst_h.at[pl.ds(wr0, WR)], didx.at[b])
            for j in range(WR):
                pltpu.async_copy(tbl.at[sidx.at[b, j]], rows.at[b, j],
                                 sems[b])

        load_issue(0, 0)
        load_issue(1, 1)

        def g_body(g, carry):
            for b in range(2):
                for j in range(WR):
                    pltpu.make_async_copy(dummy, rows.at[b, j], sems[b]).wait()
                for j in range(WR):
                    pltpu.sync_copy(rows.at[b, j], acc.at[didx.at[b, j]],
                                    add=True)

                @pl.when(g < NG - 1)
                def _():
                    load_issue(2 * g + b + 2, b)
            return carry

        lax.fori_loop(0, NG, g_body, 0)
        plsc.subcore_barrier()
        pltpu.sync_copy(acc.at[pl.ds(r0, RPT)], agg_h.at[c, pl.ds(r0, RPT)])

        if with_count:
            plsc.subcore_barrier()
            pltpu.sync_copy(zeros_h.at[pl.ds(r0, RPT)], acc.at[pl.ds(r0, RPT)])
            plsc.subcore_barrier()
            wid = s * NC + c
            crow0 = wid * CHRC

            def c_body(w, carry):
                wr0 = crow0 + w * WRC
                pltpu.sync_copy(dst_h.at[pl.ds(wr0, WRC)],
                                didx.at[0, pl.ds(0, WRC)])
                for j in range(WRC):
                    pltpu.sync_copy(ones, acc.at[didx.at[0, j]], add=True)
                return carry

            lax.fori_loop(0, NWINC, c_body, 0)
            plsc.subcore_barrier()
            pltpu.sync_copy(acc.at[pl.ds(r0, RPT)], cnt_h.at[c, pl.ds(r0, RPT)])

    fn = pl.kernel(
        body,
        out_type=(
            jax.ShapeDtypeStruct((NC, N2, 16), jnp.float32),
            jax.ShapeDtypeStruct((NC, N2, 16) if with_count else (NC, 8, 16),
                                 jnp.float32),
        ),
        mesh=plsc.VectorSubcoreMesh(core_axis_name="c", subcore_axis_name="s"),
        compiler_params=pltpu.CompilerParams(use_tc_tiling_on_sc=False),
        scratch_types=[
            pltpu.VMEM((2, WR, 128), jnp.int32),
            pltpu.VMEM((2, WR, 128), jnp.int32),
            pltpu.VMEM((2, WR, 128, 16), jnp.float32),
            pltpu.VMEM((128, 16), jnp.float32),
            pltpu.VMEM_SHARED((N2, 16), jnp.float32),
            pltpu.SemaphoreType.DMA,
            pltpu.SemaphoreType.DMA,
        ],
    )
    return fn(table, src2, dst2, zeros2)


GRID = 4
BR = N2P8 // GRID
BLKN = BR * 8


def _np_kron_eye(w, k):
    r, c = w.shape
    out = jnp.zeros((k * r, k * c), jnp.float32)
    for a in range(k):
        out = lax.dynamic_update_slice(out, w, (a * r, a * c))
    return out


def _sel_count():
    s = np.zeros((128, 128), np.float32)
    for a in range(8):
        s[16 * a + 8, 16 * a:16 * a + 16] = 1.0
    return s


def _sel_gsum():
    g = np.zeros((256, 256), np.float32)
    for a in range(8):
        g[32 * a:32 * a + 32, 32 * a:32 * a + 32] = 1.0
    return g


def _sel_half(which):
    p = np.zeros((256, 128), np.float32)
    for a in range(8):
        for k in range(16):
            p[32 * a + 16 * which + k, 16 * a + k] = 1.0
    return p


_S_CNT = _sel_count()
_G_SUM = _sel_gsum()
_P_HALF0 = _sel_half(0)
_P_HALF1 = _sel_half(1)


def _sel_pack(which):
    t = np.zeros((128, 128), np.float32)
    for m in range(8):
        for k in range(IN_DIM):
            t[64 * which + 8 * m + k, 16 * m + k] = 1.0
    return t


_T_PACK0 = _sel_pack(0)
_T_PACK1 = _sel_pack(1)
_ONES_LANE = np.zeros((1, 128), np.float32)
for _m in range(8):
    _ONES_LANE[0, 16 * _m + IN_DIM] = 1.0


def _prep_x(x):
    x_lin = x.reshape(N // 16, 128)
    a = x_lin @ jnp.asarray(_T_PACK0)
    b = x_lin @ jnp.asarray(_T_PACK1)
    pk = jnp.stack([a, b], axis=1).reshape(NP8, 128) + jnp.asarray(_ONES_LANE)
    return jnp.concatenate(
        [pk, jnp.zeros((N2P8 - NP8, 128), jnp.float32)], axis=0)


def _dense1(parts_pk, x_pk, kwl, kwr, kb, gsum, scnt, ph0, ph1):
    def body(p_ref, x_ref, kwl_ref, kwr_ref, kb_ref, g_ref, s_ref,
             ph0_ref, ph1_ref, h_ref, cnt_ref):
        agg = p_ref[0] + p_ref[1]
        cnt = jnp.dot(agg, s_ref[...], preferred_element_type=jnp.float32)
        mean = agg / jnp.maximum(cnt, 1.0)
        out = (jnp.dot(mean, kwl_ref[...], preferred_element_type=jnp.float32)
               + jnp.dot(x_ref[...], kwr_ref[...],
                         preferred_element_type=jnp.float32)
               + kb_ref[...])
        ss = jnp.dot(out * out, g_ref[...], preferred_element_type=jnp.float32)
        out = out / jnp.maximum(jnp.sqrt(ss), 1e-12)
        out = jnp.tanh(out)
        h_ref[0] = jnp.dot(out, ph0_ref[...],
                           preferred_element_type=jnp.float32)
        h_ref[1] = jnp.dot(out, ph1_ref[...],
                           preferred_element_type=jnp.float32)
        cnt_ref[...] = cnt

    return pl.pallas_call(
        body,
        grid=(GRID,),
        in_specs=[
            pl.BlockSpec((2, BR, 128), lambda i: (0, i, 0)),
            pl.BlockSpec((BR, 128), lambda i: (i, 0)),
            pl.BlockSpec((128, 256), lambda i: (0, 0)),
            pl.BlockSpec((128, 256), lambda i: (0, 0)),
            pl.BlockSpec((1, 256), lambda i: (0, 0)),
            pl.BlockSpec((256, 256), lambda i: (0, 0)),
            pl.BlockSpec((128, 128), lambda i: (0, 0)),
            pl.BlockSpec((256, 128), lambda i: (0, 0)),
            pl.BlockSpec((256, 128), lambda i: (0, 0)),
        ],
        out_specs=[
            pl.BlockSpec((2, BR, 128), lambda i: (0, i, 0)),
            pl.BlockSpec((BR, 128), lambda i: (i, 0)),
        ],
        out_shape=[
            jax.ShapeDtypeStruct((2, N2P8, 128), jnp.float32),
            jax.ShapeDtypeStruct((N2P8, 128), jnp.float32),
        ],
    )(parts_pk, x_pk, kwl, kwr, kb, gsum, scnt, ph0, ph1)


def _dense23(agg_pk, cnt_pk, h_pk, kwl_a, kwl_b, kwr_a, kwr_b, kb, gsum,
             ph0, ph1, final):
    cnt_is_split = cnt_pk.ndim == 3

    def body(agg_ref, cnt_ref, h_ref, kwla_ref, kwlb_ref, kwra_ref, kwrb_ref,
             kb_ref, g_ref, ph0_ref, ph1_ref, o_ref):
        if cnt_is_split:
            cnt = cnt_ref[0] + cnt_ref[1]
        else:
            cnt = cnt_ref[...]
        cnt = jnp.maximum(cnt, 1.0)
        mean0 = agg_ref[0] / cnt
        mean1 = agg_ref[1] / cnt
        out = (jnp.dot(mean0, kwla_ref[...], preferred_element_type=jnp.float32)
               + jnp.dot(mean1, kwlb_ref[...],
                         preferred_element_type=jnp.float32)
               + jnp.dot(h_ref[0], kwra_ref[...],
                         preferred_element_type=jnp.float32)
               + jnp.dot(h_ref[1], kwrb_ref[...],
                         preferred_element_type=jnp.float32)
               + kb_ref[...])
        ss = jnp.dot(out * out, g_ref[...], preferred_element_type=jnp.float32)
        out = out / jnp.maximum(jnp.sqrt(ss), 1e-12)
        out = jnp.tanh(out)
        if final:
            o_ref[...] = out
        else:
            o_ref[0] = jnp.dot(out, ph0_ref[...],
                               preferred_element_type=jnp.float32)
            o_ref[1] = jnp.dot(out, ph1_ref[...],
                               preferred_element_type=jnp.float32)

    cnt_spec = (pl.BlockSpec((2, BR, 128), lambda i: (0, i, 0)) if cnt_is_split
                else pl.BlockSpec((BR, 128), lambda i: (i, 0)))
    out_spec = (pl.BlockSpec((BR, 256), lambda i: (i, 0)) if final
                else pl.BlockSpec((2, BR, 128), lambda i: (0, i, 0)))
    out_shape = (jax.ShapeDtypeStruct((N2P8, 256), jnp.float32) if final
                 else jax.ShapeDtypeStruct((2, N2P8, 128), jnp.float32))
    return pl.pallas_call(
        body,
        grid=(GRID,),
        in_specs=[
            pl.BlockSpec((2, BR, 128), lambda i: (0, i, 0)),
            cnt_spec,
            pl.BlockSpec((2, BR, 128), lambda i: (0, i, 0)),
            pl.BlockSpec((128, 256), lambda i: (0, 0)),
            pl.BlockSpec((128, 256), lambda i: (0, 0)),
            pl.BlockSpec((128, 256), lambda i: (0, 0)),
            pl.BlockSpec((128, 256), lambda i: (0, 0)),
            pl.BlockSpec((1, 256), lambda i: (0, 0)),
            pl.BlockSpec((256, 256), lambda i: (0, 0)),
            pl.BlockSpec((256, 128), lambda i: (0, 0)),
            pl.BlockSpec((256, 128), lambda i: (0, 0)),
        ],
        out_specs=out_spec,
        out_shape=out_shape,
    )(agg_pk, cnt_pk, h_pk, kwl_a, kwl_b, kwr_a, kwr_b, kb, gsum, ph0, ph1)


def _pad_consts(e, ep):
    npad = ep - e
    pad_src = ((np.arange(npad, dtype=np.int64) * 97) % N).astype(np.int32)
    pad_dst = (N + (np.arange(npad, dtype=np.int64) % PAD_ROWS)).astype(
        np.int32)
    return pad_src, pad_dst


_PAD1 = _pad_consts(E1, E1P)
_PAD2 = _pad_consts(E2, E2P)


def _pad_edges(edge_index, ep, pads):
    pad_src, pad_dst = pads
    src2 = jnp.concatenate([edge_index[0], pad_src]).reshape(ep // 128, 128)
    dst2 = jnp.concatenate([edge_index[1], pad_dst]).reshape(ep // 128, 128)
    return src2, dst2


def kernel(x, edge_index_connections, edge_index_destinations,
           W_l1, W_r1, b1, W_l2, W_r2, b2, W_l3, W_r3, b3):
    f32 = jnp.float32
    src1, dst1 = _pad_edges(edge_index_connections, E1P, _PAD1)
    src2, dst2 = _pad_edges(edge_index_destinations, E2P, _PAD2)

    zeros2 = jnp.zeros((N2, 16), f32)
    gsum = jnp.asarray(_G_SUM)
    scnt = jnp.asarray(_S_CNT)
    ph0 = jnp.asarray(_P_HALF0)
    ph1 = jnp.asarray(_P_HALF1)

    wl1 = jnp.zeros((16, HIDDEN), f32).at[:IN_DIM].set(W_l1)
    wr1 = jnp.zeros((16, HIDDEN), f32).at[:IN_DIM].set(W_r1)
    kwl1 = _np_kron_eye(wl1, 8)
    kwr1 = _np_kron_eye(wr1, 8)
    kwl2_a = _np_kron_eye(W_l2[:16], 8)
    kwl2_b = _np_kron_eye(W_l2[16:], 8)
    kwr2_a = _np_kron_eye(W_r2[:16], 8)
    kwr2_b = _np_kron_eye(W_r2[16:], 8)
    kwl3_a = _np_kron_eye(W_l3[:16], 8)
    kwl3_b = _np_kron_eye(W_l3[16:], 8)
    kwr3_a = _np_kron_eye(W_r3[:16], 8)
    kwr3_b = _np_kron_eye(W_r3[16:], 8)
    kb1 = jnp.tile(b1, 8).reshape(1, 256)
    kb2 = jnp.tile(b2, 8).reshape(1, 256)
    kb3 = jnp.tile(b3, 8).reshape(1, 256)

    x_pk = _prep_x(x)
    x_tbl = x_pk.reshape(N2, 16)

    parts = _agg_edge_split(x_tbl, src1, dst1, zeros2)
    h1, cnt1 = _dense1(parts.reshape(NC, N2P8, 128), x_pk,
                       kwl1, kwr1, kb1, gsum, scnt, ph0, ph1)

    agg2, cnt2p = _agg_col_split(h1.reshape(NC, N2, 16), src2, dst2, zeros2,
                                 E2P, True)
    h2 = _dense23(agg2.reshape(NC, N2P8, 128), cnt2p.reshape(NC, N2P8, 128),
                  h1, kwl2_a, kwl2_b, kwr2_a, kwr2_b, kb2, gsum, ph0, ph1,
                  False)

    agg3, _ = _agg_col_split(h2.reshape(NC, N2, 16), src1, dst1, zeros2,
                             E1P, False)
    out = _dense23(agg3.reshape(NC, N2P8, 128), cnt1, h2,
                   kwl3_a, kwl3_b, kwr3_a, kwr3_b, kb3, gsum, ph0, ph1,
                   True)
    return out[:NP8].reshape(N, HIDDEN)
